# speedup vs baseline: 1.0000x; 1.0000x over previous
import jax, jax.numpy as jnp
from jax.experimental import pallas as pl

K = 20


def _bn(x, g, b):
    axes = tuple(i for i in range(x.ndim) if i != 1)
    m = jnp.mean(x, axis=axes, keepdims=True)
    v = jnp.var(x, axis=axes, keepdims=True)
    shape = [1] * x.ndim
    shape[1] = x.shape[1]
    return (x - m) / jnp.sqrt(v + 1e-5) * g.reshape(shape) + b.reshape(shape)


def _c1(x, w, b=None):
    y = jnp.einsum('oc,bcn->bon', w, x)
    if b is not None:
        y = y + b[None, :, None]
    return y


def _c2(x, w, b):
    return jnp.einsum('oc,bcnk->bonk', w, x) + b[None, :, None, None]


def _gather(x, idx):
    xt = jnp.transpose(x, (0, 2, 1))
    f = jax.vmap(lambda a, i: a[i])(xt, idx)
    return jnp.transpose(f, (0, 3, 1, 2))


def _knn(x, k):
    inner = -2.0 * jnp.einsum('bcn,bcm->bnm', x, x)
    xx = jnp.sum(x ** 2, axis=1, keepdims=True)
    pd = -xx - inner - jnp.transpose(xx, (0, 2, 1))
    return jax.lax.top_k(pd, k)[1]


def _sam(p, x, idx, xyz):
    q = _c1(x, p['w1'], p['b1'])
    kk = _c1(x, p['w2'], p['b2'])
    v = _c1(x, p['w3'], p['b3'])
    pos = _c1(xyz, p['w4'], p['b4'])
    score = _gather(kk, idx) - q[:, :, :, None]
    values = _gather(v, idx)
    positional = _gather(pos, idx) - pos[:, :, :, None]
    score = jnp.concatenate([score, positional], axis=1)
    values = jnp.concatenate([values, positional], axis=1)
    score = jax.nn.relu(_bn(score, p['g1'], p['be1']))
    values = jax.nn.relu(_bn(values, p['g2'], p['be2']))
    score = jax.nn.softmax(_c2(score, p['ws'], p['bs']), axis=-1)
    values = _c2(values, p['wv'], p['bv'])
    return jnp.sum(score * values, axis=-1)


def _bneck(p, x, idx, xyz):
    out = jax.nn.relu(_bn(x, p['g1'], p['b1']))
    out = jax.nn.relu(_bn(_sam(p['sam'], out, idx, xyz), p['g2'], p['b2']))
    out = _c1(out, p['wc'], p['bc'])
    return out + x


def _id_kernel(x_ref, o_ref):
    o_ref[...] = x_ref[...]


def kernel(x, params):
    p = params
    xyz = x
    h = jax.nn.leaky_relu(_bn(_c1(x, p['w_in']), p['g_in'], p['b_in']), 0.2)
    idx = _knn(h, K)
    h = _bneck(p['sa1'], _c1(h, p['fc1_w'], p['fc1_b']), idx, xyz)
    x1 = h
    idx = _knn(h, K)
    h = _bneck(p['sa2'], _c1(h, p['fc2_w'], p['fc2_b']), idx, xyz)
    x2 = h
    idx = _knn(h, K)
    h = _bneck(p['sa3'], _c1(h, p['fc3_w'], p['fc3_b']), idx, xyz)
    x3 = h
    idx = _knn(h, K)
    h = _bneck(p['sa4'], _c1(h, p['fc4_w'], p['fc4_b']), idx, xyz)
    x4 = h
    h = jnp.concatenate([x1, x2, x3, x4], axis=1)
    h = jax.nn.relu(_bn(_c1(h, p['fc5_w']), p['g5'], p['b5']))
    h = jnp.max(h, axis=-1)
    h = jax.nn.relu(_bn(h @ p['l1_w'].T, p['g6'], p['b6']))
    h = jax.nn.relu(_bn(h @ p['l2_w'].T + p['l2_b'], p['g7'], p['b7']))
    out = h @ p['l3_w'].T + p['l3_b']
    return pl.pallas_call(
        _id_kernel,
        out_shape=jax.ShapeDtypeStruct(out.shape, out.dtype),
    )(out)


# speedup vs baseline: 11.1785x; 11.1782x over previous
import functools

import jax
import jax.numpy as jnp
from jax import lax
from jax.experimental import pallas as pl
from jax.experimental.pallas import tpu as pltpu
from jax.experimental.pallas import tpu_sc as plsc

K = 20
B = 8
N = 1024
BN = B * N
NIDX = BN * K
NW = 32
BPW = NIDX // NW


def _knn_body(h_ref, ht_ref, o_ref):
    hb = h_ref[...]
    htb = ht_ref[...]
    g2 = jnp.dot(hb, htb, preferred_element_type=jnp.float32)
    xxc = jnp.sum(hb * hb, axis=1, keepdims=True)
    xxr = jnp.sum(htb * htb, axis=0, keepdims=True)
    pd = 2.0 * g2 - xxc - xxr
    li = lax.broadcasted_iota(jnp.int32, (N, N), 1)
    cols = []
    for _ in range(K):
        m = jnp.max(pd, axis=1, keepdims=True)
        idxk = jnp.min(jnp.where(pd == m, li, N), axis=1, keepdims=True)
        cols.append(idxk)
        pd = jnp.where(li == idxk, -jnp.inf, pd)
    cols.append(jnp.zeros((N, 32 - K), jnp.int32))
    o_ref[...] = jnp.concatenate(cols, axis=1)[None]


def _knn(x, k):
    c = x.shape[1]
    hN = jnp.transpose(x, (0, 2, 1)).reshape(BN, c)
    idx3 = pl.pallas_call(
        _knn_body,
        grid=(B,),
        in_specs=[
            pl.BlockSpec((N, c), lambda b: (b, 0)),
            pl.BlockSpec((c, N), lambda b: (0, b)),
        ],
        out_specs=pl.BlockSpec((1, N, 32), lambda b: (b, 0, 0)),
        out_shape=jax.ShapeDtypeStruct((B, N, 32), jnp.int32),
    )(hN, jnp.transpose(hN))
    return idx3[:, :, :K]


def _make_sc_gather(D, CH):
    nch = BPW // CH
    mesh = plsc.VectorSubcoreMesh(core_axis_name="c", subcore_axis_name="s")

    @functools.partial(
        pl.kernel,
        mesh=mesh,
        out_type=jax.ShapeDtypeStruct((NIDX, D), jnp.float32),
        scratch_types=[
            pltpu.VMEM((CH,), jnp.int32),
            pltpu.VMEM((CH, D), jnp.float32),
            pltpu.SemaphoreType.DMA,
        ],
    )
    def k(table_hbm, idx_hbm, out_hbm, idx_v, rows_v, sem):
        wid = lax.axis_index("s") * 2 + lax.axis_index("c")
        base = wid * BPW

        def body(i, carry):
            off = base + i * CH
            pltpu.sync_copy(idx_hbm.at[pl.ds(off, CH)], idx_v)
            pltpu.async_copy(table_hbm.at[idx_v], rows_v, sem).wait()
            pltpu.sync_copy(rows_v, out_hbm.at[pl.ds(off, CH)])
            return carry

        lax.fori_loop(0, nch, body, 0)

    return k


_SC_CACHE = {}


def _sc_gather(table, idxf):
    D = table.shape[1]
    CH = max(64, (32768 // D) // 64 * 64)
    key = (D, CH)
    if key not in _SC_CACHE:
        _SC_CACHE[key] = _make_sc_gather(D, CH)
    return _SC_CACHE[key](table, idxf)


def _gather3(kk, v, pos, idx):
    mid = kk.shape[1]
    dpad = (-3 * mid) % 128
    parts = [jnp.transpose(kk, (0, 2, 1)).reshape(BN, mid),
             jnp.transpose(v, (0, 2, 1)).reshape(BN, mid),
             jnp.transpose(pos, (0, 2, 1)).reshape(BN, mid)]
    if dpad:
        parts.append(jnp.zeros((BN, dpad), jnp.float32))
    tab = jnp.concatenate(parts, axis=1)
    offs = (jnp.arange(B, dtype=jnp.int32) * N)[:, None, None]
    idxf = (idx.astype(jnp.int32) + offs).reshape(NIDX)
    g = _sc_gather(tab, idxf)[:, :3 * mid]
    g = jnp.transpose(g.reshape(B, N, K, 3 * mid), (0, 3, 1, 2))
    return g[:, 0:mid], g[:, mid:2 * mid], g[:, 2 * mid:3 * mid]


def _bn(x, g, b):
    axes = tuple(i for i in range(x.ndim) if i != 1)
    m = jnp.mean(x, axis=axes, keepdims=True)
    v = jnp.var(x, axis=axes, keepdims=True)
    shape = [1] * x.ndim
    shape[1] = x.shape[1]
    return (x - m) / jnp.sqrt(v + 1e-5) * g.reshape(shape) + b.reshape(shape)


def _c1(x, w, b=None):
    y = jnp.einsum('oc,bcn->bon', w, x)
    if b is not None:
        y = y + b[None, :, None]
    return y


def _c2(x, w, b):
    return jnp.einsum('oc,bcnk->bonk', w, x) + b[None, :, None, None]


def _sam(p, x, idx, xyz):
    q = _c1(x, p['w1'], p['b1'])
    kk = _c1(x, p['w2'], p['b2'])
    v = _c1(x, p['w3'], p['b3'])
    pos = _c1(xyz, p['w4'], p['b4'])
    gkk, gv, gpos = _gather3(kk, v, pos, idx)
    score = gkk - q[:, :, :, None]
    values = gv
    positional = gpos - pos[:, :, :, None]
    score = jnp.concatenate([score, positional], axis=1)
    values = jnp.concatenate([values, positional], axis=1)
    score = jax.nn.relu(_bn(score, p['g1'], p['be1']))
    values = jax.nn.relu(_bn(values, p['g2'], p['be2']))
    score = jax.nn.softmax(_c2(score, p['ws'], p['bs']), axis=-1)
    values = _c2(values, p['wv'], p['bv'])
    return jnp.sum(score * values, axis=-1)


def _bneck(p, x, idx, xyz):
    out = jax.nn.relu(_bn(x, p['g1'], p['b1']))
    out = jax.nn.relu(_bn(_sam(p['sam'], out, idx, xyz), p['g2'], p['b2']))
    out = _c1(out, p['wc'], p['bc'])
    return out + x


def kernel(x, params):
    p = params
    xyz = x
    h = jax.nn.leaky_relu(_bn(_c1(x, p['w_in']), p['g_in'], p['b_in']), 0.2)
    idx = _knn(h, K)
    h = _bneck(p['sa1'], _c1(h, p['fc1_w'], p['fc1_b']), idx, xyz)
    x1 = h
    idx = _knn(h, K)
    h = _bneck(p['sa2'], _c1(h, p['fc2_w'], p['fc2_b']), idx, xyz)
    x2 = h
    idx = _knn(h, K)
    h = _bneck(p['sa3'], _c1(h, p['fc3_w'], p['fc3_b']), idx, xyz)
    x3 = h
    idx = _knn(h, K)
    h = _bneck(p['sa4'], _c1(h, p['fc4_w'], p['fc4_b']), idx, xyz)
    x4 = h
    h = jnp.concatenate([x1, x2, x3, x4], axis=1)
    h = jax.nn.relu(_bn(_c1(h, p['fc5_w']), p['g5'], p['b5']))
    h = jnp.max(h, axis=-1)
    h = jax.nn.relu(_bn(h @ p['l1_w'].T, p['g6'], p['b6']))
    h = jax.nn.relu(_bn(h @ p['l2_w'].T + p['l2_b'], p['g7'], p['b7']))
    return h @ p['l3_w'].T + p['l3_b']


# speedup vs baseline: 11.7085x; 1.0474x over previous
import functools

import jax
import jax.numpy as jnp
from jax import lax
from jax.experimental import pallas as pl
from jax.experimental.pallas import tpu as pltpu
from jax.experimental.pallas import tpu_sc as plsc

K = 20
B = 8
N = 1024
BN = B * N
NIDX = BN * K
NW = 32
BPW = NIDX // NW


def _knn_body(h_ref, ht_ref, o_ref):
    hb = h_ref[...]
    htb = ht_ref[...]
    g2 = jnp.dot(hb, htb, preferred_element_type=jnp.float32)
    xxc = jnp.sum(hb * hb, axis=1, keepdims=True)
    xxr = jnp.sum(htb * htb, axis=0, keepdims=True)
    pd = 2.0 * g2 - xxc - xxr
    li = lax.broadcasted_iota(jnp.int32, (N, N), 1)
    cols = []
    for _ in range(K):
        m = jnp.max(pd, axis=1, keepdims=True)
        idxk = jnp.min(jnp.where(pd == m, li, N), axis=1, keepdims=True)
        cols.append(idxk)
        pd = jnp.where(li == idxk, -jnp.inf, pd)
    cols.append(jnp.zeros((N, 32 - K), jnp.int32))
    o_ref[...] = jnp.concatenate(cols, axis=1)[None]


def _knn(x, k):
    c = x.shape[1]
    hN = jnp.transpose(x, (0, 2, 1)).reshape(BN, c)
    idx3 = pl.pallas_call(
        _knn_body,
        grid=(B,),
        in_specs=[
            pl.BlockSpec((N, c), lambda b: (b, 0)),
            pl.BlockSpec((c, N), lambda b: (0, b)),
        ],
        out_specs=pl.BlockSpec((1, N, 32), lambda b: (b, 0, 0)),
        out_shape=jax.ShapeDtypeStruct((B, N, 32), jnp.int32),
    )(hN, jnp.transpose(hN))
    return idx3[:, :, :K]


def _make_sc_gather(D, CH):
    nch = BPW // CH
    mesh = plsc.VectorSubcoreMesh(core_axis_name="c", subcore_axis_name="s")

    @functools.partial(
        pl.kernel,
        mesh=mesh,
        out_type=jax.ShapeDtypeStruct((NIDX, D), jnp.float32),
        scratch_types=[
            pltpu.VMEM((CH,), jnp.int32),
            pltpu.VMEM((CH,), jnp.int32),
            pltpu.VMEM((CH, D), jnp.float32),
            pltpu.VMEM((CH, D), jnp.float32),
            pltpu.SemaphoreType.DMA,
            pltpu.SemaphoreType.DMA,
            pltpu.SemaphoreType.DMA,
            pltpu.SemaphoreType.DMA,
            pltpu.SemaphoreType.DMA,
            pltpu.SemaphoreType.DMA,
        ],
    )
    def k(table_hbm, idx_hbm, out_hbm, ia, ib, ra, rb, sia, sib, sga, sgb,
          soa, sob):
        wid = lax.axis_index("s") * 2 + lax.axis_index("c")
        base = wid * BPW

        def idx_at(i):
            return idx_hbm.at[pl.ds(base + i * CH, CH)]

        def out_at(i):
            return out_hbm.at[pl.ds(base + i * CH, CH)]

        pltpu.async_copy(idx_at(0), ia, sia).wait()
        pltpu.async_copy(table_hbm.at[ia], ra, sga)

        def body(j, carry):
            i0 = 2 * j
            i1 = i0 + 1
            inx = jnp.minimum(i0 + 2, nch - 1)
            pltpu.async_copy(idx_at(i1), ib, sib).wait()
            pltpu.async_copy(table_hbm.at[ib], rb, sgb)
            pltpu.make_async_copy(table_hbm.at[ia], ra, sga).wait()
            pltpu.async_copy(ra, out_at(i0), soa)
            pltpu.make_async_copy(ra, out_at(i0), soa).wait()
            pltpu.async_copy(idx_at(inx), ia, sia).wait()
            pltpu.async_copy(table_hbm.at[ia], ra, sga)
            pltpu.make_async_copy(table_hbm.at[ib], rb, sgb).wait()
            pltpu.async_copy(rb, out_at(i1), sob)
            pltpu.make_async_copy(rb, out_at(i1), sob).wait()
            return carry

        lax.fori_loop(0, nch // 2, body, 0)
        pltpu.make_async_copy(table_hbm.at[ia], ra, sga).wait()

    return k


_SC_CACHE = {}


def _sc_gather(table, idxf):
    D = table.shape[1]
    CH = {128: 320, 256: 160, 384: 128, 512: 64}[D]
    key = (D, CH)
    if key not in _SC_CACHE:
        _SC_CACHE[key] = _make_sc_gather(D, CH)
    return _SC_CACHE[key](table, idxf)


def _gather3(kk, v, pos, idx):
    mid = kk.shape[1]
    dpad = (-3 * mid) % 128
    parts = [jnp.transpose(kk, (0, 2, 1)).reshape(BN, mid),
             jnp.transpose(v, (0, 2, 1)).reshape(BN, mid),
             jnp.transpose(pos, (0, 2, 1)).reshape(BN, mid)]
    if dpad:
        parts.append(jnp.zeros((BN, dpad), jnp.float32))
    tab = jnp.concatenate(parts, axis=1)
    offs = (jnp.arange(B, dtype=jnp.int32) * N)[:, None, None]
    idxf = (idx.astype(jnp.int32) + offs).reshape(NIDX)
    g = _sc_gather(tab, idxf)[:, :3 * mid]
    g = jnp.transpose(g.reshape(B, N, K, 3 * mid), (0, 3, 1, 2))
    return g[:, 0:mid], g[:, mid:2 * mid], g[:, 2 * mid:3 * mid]


def _bn(x, g, b):
    axes = tuple(i for i in range(x.ndim) if i != 1)
    m = jnp.mean(x, axis=axes, keepdims=True)
    v = jnp.var(x, axis=axes, keepdims=True)
    shape = [1] * x.ndim
    shape[1] = x.shape[1]
    return (x - m) / jnp.sqrt(v + 1e-5) * g.reshape(shape) + b.reshape(shape)


def _c1(x, w, b=None):
    y = jnp.einsum('oc,bcn->bon', w, x)
    if b is not None:
        y = y + b[None, :, None]
    return y


def _c2(x, w, b):
    return jnp.einsum('oc,bcnk->bonk', w, x) + b[None, :, None, None]


def _sam(p, x, idx, xyz):
    q = _c1(x, p['w1'], p['b1'])
    kk = _c1(x, p['w2'], p['b2'])
    v = _c1(x, p['w3'], p['b3'])
    pos = _c1(xyz, p['w4'], p['b4'])
    gkk, gv, gpos = _gather3(kk, v, pos, idx)
    score = gkk - q[:, :, :, None]
    values = gv
    positional = gpos - pos[:, :, :, None]
    score = jnp.concatenate([score, positional], axis=1)
    values = jnp.concatenate([values, positional], axis=1)
    score = jax.nn.relu(_bn(score, p['g1'], p['be1']))
    values = jax.nn.relu(_bn(values, p['g2'], p['be2']))
    score = jax.nn.softmax(_c2(score, p['ws'], p['bs']), axis=-1)
    values = _c2(values, p['wv'], p['bv'])
    return jnp.sum(score * values, axis=-1)


def _bneck(p, x, idx, xyz):
    out = jax.nn.relu(_bn(x, p['g1'], p['b1']))
    out = jax.nn.relu(_bn(_sam(p['sam'], out, idx, xyz), p['g2'], p['b2']))
    out = _c1(out, p['wc'], p['bc'])
    return out + x


def kernel(x, params):
    p = params
    xyz = x
    h = jax.nn.leaky_relu(_bn(_c1(x, p['w_in']), p['g_in'], p['b_in']), 0.2)
    idx = _knn(h, K)
    h = _bneck(p['sa1'], _c1(h, p['fc1_w'], p['fc1_b']), idx, xyz)
    x1 = h
    idx = _knn(h, K)
    h = _bneck(p['sa2'], _c1(h, p['fc2_w'], p['fc2_b']), idx, xyz)
    x2 = h
    idx = _knn(h, K)
    h = _bneck(p['sa3'], _c1(h, p['fc3_w'], p['fc3_b']), idx, xyz)
    x3 = h
    idx = _knn(h, K)
    h = _bneck(p['sa4'], _c1(h, p['fc4_w'], p['fc4_b']), idx, xyz)
    x4 = h
    h = jnp.concatenate([x1, x2, x3, x4], axis=1)
    h = jax.nn.relu(_bn(_c1(h, p['fc5_w']), p['g5'], p['b5']))
    h = jnp.max(h, axis=-1)
    h = jax.nn.relu(_bn(h @ p['l1_w'].T, p['g6'], p['b6']))
    h = jax.nn.relu(_bn(h @ p['l2_w'].T + p['l2_b'], p['g7'], p['b7']))
    return h @ p['l3_w'].T + p['l3_b']


# speedup vs baseline: 11.7199x; 1.0010x over previous
import functools

import jax
import jax.numpy as jnp
from jax import lax
from jax.experimental import pallas as pl
from jax.experimental.pallas import tpu as pltpu
from jax.experimental.pallas import tpu_sc as plsc

K = 20
B = 8
N = 1024
BN = B * N
NIDX = BN * K
NW = 32
BPW = NIDX // NW


def _knn_body(h_ref, ht_ref, o_ref):
    hb = h_ref[...]
    htb = ht_ref[...]
    g2 = jnp.dot(hb, htb, preferred_element_type=jnp.float32)
    xxc = jnp.sum(hb * hb, axis=1, keepdims=True)
    xxr = jnp.sum(htb * htb, axis=0, keepdims=True)
    pd = 2.0 * g2 - xxc - xxr
    li = lax.broadcasted_iota(jnp.int32, (N, N), 1)
    cols = []
    for _ in range(K):
        m = jnp.max(pd, axis=1, keepdims=True)
        idxk = jnp.min(jnp.where(pd == m, li, N), axis=1, keepdims=True)
        cols.append(idxk)
        pd = jnp.where(li == idxk, -jnp.inf, pd)
    cols.append(jnp.zeros((N, 32 - K), jnp.int32))
    o_ref[...] = jnp.concatenate(cols, axis=1)[None]


def _knn(x, k):
    c = x.shape[1]
    hN = jnp.transpose(x, (0, 2, 1)).reshape(BN, c)
    idx3 = pl.pallas_call(
        _knn_body,
        grid=(B,),
        in_specs=[
            pl.BlockSpec((N, c), lambda b: (b, 0)),
            pl.BlockSpec((c, N), lambda b: (0, b)),
        ],
        out_specs=pl.BlockSpec((1, N, 32), lambda b: (b, 0, 0)),
        out_shape=jax.ShapeDtypeStruct((B, N, 32), jnp.int32),
    )(hN, jnp.transpose(hN))
    return idx3[:, :, :K]


def _make_sc_gather(D, CH):
    nch = BPW // CH
    mesh = plsc.VectorSubcoreMesh(core_axis_name="c", subcore_axis_name="s")

    @functools.partial(
        pl.kernel,
        mesh=mesh,
        out_type=jax.ShapeDtypeStruct((NIDX, D), jnp.float32),
        scratch_types=[
            pltpu.VMEM((BPW,), jnp.int32),
            pltpu.VMEM((CH, D), jnp.float32),
            pltpu.VMEM((CH, D), jnp.float32),
            pltpu.SemaphoreType.DMA,
            pltpu.SemaphoreType.DMA,
            pltpu.SemaphoreType.DMA,
            pltpu.SemaphoreType.DMA,
        ],
    )
    def k(table_hbm, idx_hbm, out_hbm, idx_v, ra, rb, sga, sgb, soa, sob):
        wid = lax.axis_index("s") * 2 + lax.axis_index("c")
        base = wid * BPW

        def idx_at(i):
            return idx_v.at[pl.ds(i * CH, CH)]

        def out_at(i):
            return out_hbm.at[pl.ds(base + i * CH, CH)]

        pltpu.sync_copy(idx_hbm.at[pl.ds(base, BPW)], idx_v)
        pltpu.async_copy(table_hbm.at[idx_at(0)], ra, sga)

        def body(j, carry):
            i0 = 2 * j
            i1 = i0 + 1
            inx = jnp.minimum(i0 + 2, nch - 1)
            pltpu.async_copy(table_hbm.at[idx_at(i1)], rb, sgb)
            pltpu.make_async_copy(table_hbm.at[idx_at(i0)], ra, sga).wait()
            pltpu.async_copy(ra, out_at(i0), soa)
            pltpu.make_async_copy(ra, out_at(i0), soa).wait()
            pltpu.async_copy(table_hbm.at[idx_at(inx)], ra, sga)
            pltpu.make_async_copy(table_hbm.at[idx_at(i1)], rb, sgb).wait()
            pltpu.async_copy(rb, out_at(i1), sob)
            pltpu.make_async_copy(rb, out_at(i1), sob).wait()
            return carry

        lax.fori_loop(0, nch // 2, body, 0)
        pltpu.make_async_copy(table_hbm.at[idx_at(nch - 1)], ra, sga).wait()

    return k


_SC_CACHE = {}


def _sc_gather(table, idxf):
    D = table.shape[1]
    CH = {128: 320, 256: 160, 384: 128, 512: 64}[D]
    key = (D, CH)
    if key not in _SC_CACHE:
        _SC_CACHE[key] = _make_sc_gather(D, CH)
    return _SC_CACHE[key](table, idxf)


def _gather3(kk, v, pos, idx):
    mid = kk.shape[1]
    dpad = (-3 * mid) % 128
    parts = [jnp.transpose(kk, (0, 2, 1)).reshape(BN, mid),
             jnp.transpose(v, (0, 2, 1)).reshape(BN, mid),
             jnp.transpose(pos, (0, 2, 1)).reshape(BN, mid)]
    if dpad:
        parts.append(jnp.zeros((BN, dpad), jnp.float32))
    tab = jnp.concatenate(parts, axis=1)
    offs = (jnp.arange(B, dtype=jnp.int32) * N)[:, None, None]
    idxf = (idx.astype(jnp.int32) + offs).reshape(NIDX)
    g = _sc_gather(tab, idxf)[:, :3 * mid]
    g = jnp.transpose(g.reshape(B, N, K, 3 * mid), (0, 3, 1, 2))
    return g[:, 0:mid], g[:, mid:2 * mid], g[:, 2 * mid:3 * mid]


def _bn(x, g, b):
    axes = tuple(i for i in range(x.ndim) if i != 1)
    m = jnp.mean(x, axis=axes, keepdims=True)
    v = jnp.var(x, axis=axes, keepdims=True)
    shape = [1] * x.ndim
    shape[1] = x.shape[1]
    return (x - m) / jnp.sqrt(v + 1e-5) * g.reshape(shape) + b.reshape(shape)


def _c1(x, w, b=None):
    y = jnp.einsum('oc,bcn->bon', w, x)
    if b is not None:
        y = y + b[None, :, None]
    return y


def _c2(x, w, b):
    return jnp.einsum('oc,bcnk->bonk', w, x) + b[None, :, None, None]


def _sam(p, x, idx, xyz):
    q = _c1(x, p['w1'], p['b1'])
    kk = _c1(x, p['w2'], p['b2'])
    v = _c1(x, p['w3'], p['b3'])
    pos = _c1(xyz, p['w4'], p['b4'])
    gkk, gv, gpos = _gather3(kk, v, pos, idx)
    score = gkk - q[:, :, :, None]
    values = gv
    positional = gpos - pos[:, :, :, None]
    score = jnp.concatenate([score, positional], axis=1)
    values = jnp.concatenate([values, positional], axis=1)
    score = jax.nn.relu(_bn(score, p['g1'], p['be1']))
    values = jax.nn.relu(_bn(values, p['g2'], p['be2']))
    score = jax.nn.softmax(_c2(score, p['ws'], p['bs']), axis=-1)
    values = _c2(values, p['wv'], p['bv'])
    return jnp.sum(score * values, axis=-1)


def _bneck(p, x, idx, xyz):
    out = jax.nn.relu(_bn(x, p['g1'], p['b1']))
    out = jax.nn.relu(_bn(_sam(p['sam'], out, idx, xyz), p['g2'], p['b2']))
    out = _c1(out, p['wc'], p['bc'])
    return out + x


def kernel(x, params):
    p = params
    xyz = x
    h = jax.nn.leaky_relu(_bn(_c1(x, p['w_in']), p['g_in'], p['b_in']), 0.2)
    idx = _knn(h, K)
    h = _bneck(p['sa1'], _c1(h, p['fc1_w'], p['fc1_b']), idx, xyz)
    x1 = h
    idx = _knn(h, K)
    h = _bneck(p['sa2'], _c1(h, p['fc2_w'], p['fc2_b']), idx, xyz)
    x2 = h
    idx = _knn(h, K)
    h = _bneck(p['sa3'], _c1(h, p['fc3_w'], p['fc3_b']), idx, xyz)
    x3 = h
    idx = _knn(h, K)
    h = _bneck(p['sa4'], _c1(h, p['fc4_w'], p['fc4_b']), idx, xyz)
    x4 = h
    h = jnp.concatenate([x1, x2, x3, x4], axis=1)
    h = jax.nn.relu(_bn(_c1(h, p['fc5_w']), p['g5'], p['b5']))
    h = jnp.max(h, axis=-1)
    h = jax.nn.relu(_bn(h @ p['l1_w'].T, p['g6'], p['b6']))
    h = jax.nn.relu(_bn(h @ p['l2_w'].T + p['l2_b'], p['g7'], p['b7']))
    return h @ p['l3_w'].T + p['l3_b']
